# baseline (speedup 1.0000x reference)
import dataclasses
import functools

import jax
import jax.numpy as jnp
from jax import lax
from jax.experimental import pallas as pl
from jax.experimental.pallas import tpu as pltpu
from jax.experimental.pallas import tpu_sc as plsc

NC = 2
NS = 16
NW = NC * NS
CH = 128

_SQRT2 = 1.4142135623730951


def _ln(x, w, b):
    mu = jnp.mean(x, axis=-1, keepdims=True)
    xc = x - mu
    var = jnp.mean(xc * xc, axis=-1, keepdims=True)
    return xc * lax.rsqrt(var + 1e-5) * w + b


def _mm_t(z, w_ref):
    return lax.dot_general(z, w_ref[...], (((1,), (1,)), ((), ())),
                           precision=lax.Precision.HIGHEST,
                           preferred_element_type=jnp.float32)


def _gelu_exact(x):
    return 0.5 * x * (1.0 + lax.erf(x / _SQRT2))


def _tc_a_body(h_ref, lw_ref, lb_ref, w_ref, zt_ref):
    z = _ln(h_ref[...], lw_ref[...], lb_ref[...])
    zt_ref[...] = _mm_t(z, w_ref)


def _tc_b_body(zt_ref, deg_ref, ztp_ref, dinv_ref):
    dv = lax.rsqrt(deg_ref[...] + 1.0)
    dinv_ref[...] = dv
    ztp_ref[...] = zt_ref[...] * dv


def _tc_c_body(aggp_ref, ztp_ref, dinv_ref, b_ref, h_ref,
               lw_ref, lb_ref, w_ref, hn_ref, ztpn_ref):
    dv = dinv_ref[...]
    s = (aggp_ref[0] + aggp_ref[1] + ztp_ref[...]) * dv + b_ref[...]
    hn = _gelu_exact(s) + h_ref[...]
    hn_ref[...] = hn
    z = _ln(hn, lw_ref[...], lb_ref[...])
    ztpn_ref[...] = _mm_t(z, w_ref) * dv


def _tc_d_body(aggp_ref, ztp_ref, dinv_ref, b_ref, h_ref,
               wp_ref, bp_ref, out_ref):
    dv = dinv_ref[...]
    s = (aggp_ref[0] + aggp_ref[1] + ztp_ref[...]) * dv + b_ref[...]
    hn = _gelu_exact(s) + h_ref[...]
    out_ref[...] = _mm_t(hn, wp_ref) + bp_ref[...]


def _row_spec(r, d):
    return pl.BlockSpec((r, d), lambda i: (i, 0))


def _bc_spec(d):
    return pl.BlockSpec((1, d), lambda i: (0, 0))


def _w_spec(d):
    return pl.BlockSpec((d, d), lambda i: (0, 0))


def _pair_spec(r, d):
    return pl.BlockSpec((2, r, d), lambda i: (0, i, 0))


def _sc_compiler_params():
    cp = pltpu.CompilerParams()
    if "needs_layout_passes" in pltpu.CompilerParams.__dataclass_fields__:
        cp = dataclasses.replace(cp, needs_layout_passes=False)
    return cp


def _sc_deg(dst2d, zerosf, *, n_rows, j_chunks):
    assert n_rows <= 128
    mesh = plsc.VectorSubcoreMesh(core_axis_name="c", subcore_axis_name="s",
                                  num_cores=NC, num_subcores=NS)

    @functools.partial(
        pl.kernel, mesh=mesh, compiler_params=_sc_compiler_params(),
        out_type=jax.ShapeDtypeStruct((NC, n_rows, 128), jnp.float32),
        scratch_types=[
            pltpu.VMEM((j_chunks, CH), jnp.int32),
            pltpu.VMEM((n_rows, 128), jnp.float32),
            pltpu.VMEM((1, n_rows), jnp.int32),
            pltpu.VMEM_SHARED((n_rows, 128), jnp.float32),
        ],
    )
    def k(dst_hbm, zf_hbm, out_hbm, dst_v, acc2d, rowidx, accs):
        cid = lax.axis_index("c")
        sid = lax.axis_index("s")
        wid = sid * NC + cid
        pltpu.sync_copy(zf_hbm, acc2d)

        @pl.when(sid == 0)
        def _():
            pltpu.sync_copy(zf_hbm, accs)

        pltpu.sync_copy(dst_hbm.at[pl.ds(wid * j_chunks, j_chunks)], dst_v)

        @pl.loop(0, n_rows // 16)
        def _(c):
            rowidx[0, pl.ds(c * 16, 16)] = lax.iota(jnp.int32, 16) + c * 16

        ones = jnp.ones((16,), jnp.float32)

        @pl.loop(0, j_chunks)
        def _(j):
            @pl.loop(0, CH // 16)
            def _(c):
                idx = dst_v[j, pl.ds(c * 16, 16)]
                plsc.addupdate_scatter(
                    acc2d, [lax.shift_right_logical(idx, 7),
                            lax.bitwise_and(idx, 127)], ones)

        plsc.subcore_barrier()
        pltpu.sync_copy(acc2d, accs.at[rowidx.at[0]], add=True)
        plsc.subcore_barrier()
        half = (n_rows // 2 // 8) * 8

        @pl.when(sid == 0)
        def _():
            pltpu.sync_copy(accs.at[pl.ds(0, half)],
                            out_hbm.at[cid].at[pl.ds(0, half)])

        @pl.when(sid == 1)
        def _():
            pltpu.sync_copy(accs.at[pl.ds(half, n_rows - half)],
                            out_hbm.at[cid].at[pl.ds(half, n_rows - half)])

    return k(dst2d, zerosf)


def _make_sc_edges(*, n_acc, n_out, d, j_chunks, active_core=None):
    rows_z = n_acc // NS
    rows_o = n_out // NS
    if active_core is None:
        jt = j_chunks
        nh = 2
    else:
        jt = j_chunks * NC
        nh = 4
    jh = jt // nh
    mesh = plsc.VectorSubcoreMesh(core_axis_name="c", subcore_axis_name="s",
                                  num_cores=NC, num_subcores=NS)

    @functools.partial(
        pl.kernel, mesh=mesh,
        out_type=jax.ShapeDtypeStruct((NC, n_out, d), jnp.float32),
        scratch_types=[
            pltpu.VMEM((jh, CH), jnp.int32),
            pltpu.VMEM((jh, CH), jnp.int32),
            pltpu.VMEM((CH, d), jnp.float32),
            pltpu.VMEM((CH, d), jnp.float32),
            pltpu.VMEM_SHARED((n_acc, d), jnp.float32),
            pltpu.SemaphoreType.DMA,
            pltpu.SemaphoreType.DMA,
        ],
    )
    def k(ztp_hbm, src_hbm, dst_hbm, zb_hbm, out_hbm,
          src_v, dst_v, gbuf0, gbuf1, acc, sem0, sem1):
        cid = lax.axis_index("c")
        sid = lax.axis_index("s")
        pltpu.sync_copy(zb_hbm.at[pl.ds(sid * rows_z, rows_z)],
                        acc.at[pl.ds(sid * rows_z, rows_z)])
        plsc.subcore_barrier()

        run = (cid < NC) if active_core is None else (cid == active_core)

        @pl.when(run)
        def _():
            for h in range(nh):
                base = ((sid * NC + cid) if active_core is None else sid) * jt \
                    + h * jh
                pltpu.sync_copy(src_hbm.at[pl.ds(base, jh)], src_v)
                pltpu.sync_copy(dst_hbm.at[pl.ds(base, jh)], dst_v)
                pltpu.async_copy(ztp_hbm.at[src_v.at[0]], gbuf0, sem0)
                pltpu.async_copy(ztp_hbm.at[src_v.at[1]], gbuf1, sem1)

                @pl.loop(0, jh // 2)
                def _(p):
                    j0 = 2 * p
                    j1 = 2 * p + 1
                    pltpu.make_async_copy(ztp_hbm.at[src_v.at[j0]],
                                          gbuf0, sem0).wait()
                    pltpu.sync_copy(gbuf0, acc.at[dst_v.at[j0]], add=True)

                    @pl.when(j0 + 2 < jh)
                    def _():
                        pltpu.async_copy(ztp_hbm.at[src_v.at[j0 + 2]],
                                         gbuf0, sem0)

                    pltpu.make_async_copy(ztp_hbm.at[src_v.at[j1]],
                                          gbuf1, sem1).wait()
                    pltpu.sync_copy(gbuf1, acc.at[dst_v.at[j1]], add=True)

                    @pl.when(j1 + 2 < jh)
                    def _():
                        pltpu.async_copy(ztp_hbm.at[src_v.at[j1 + 2]],
                                         gbuf1, sem1)

        plsc.subcore_barrier()
        pltpu.sync_copy(acc.at[pl.ds(sid * rows_o, rows_o)],
                        out_hbm.at[cid].at[pl.ds(sid * rows_o, rows_o)])

    return k


def kernel(x, edge_index, emb,
           ln_w_0, ln_b_0, W_0, b_0,
           ln_w_1, ln_b_1, W_1, b_1,
           ln_w_2, ln_b_2, W_2, b_2,
           Wp, bp):
    n, d = emb.shape
    e = edge_index.shape[1]

    src = edge_index[0]
    dst = edge_index[1]
    j_chunks = -(-(-(-e // (NW * CH))) // 16) * 16
    e_pad = j_chunks * NW * CH
    pad = e_pad - e
    src_p, dst_p = src, dst
    if pad:
        src_p = jnp.concatenate([src, jnp.zeros((pad,), jnp.int32)])
        dst_p = jnp.concatenate([dst, jnp.full((pad,), n, jnp.int32)])
    src2d = src_p.reshape(NW * j_chunks, CH)
    dst2d = dst_p.reshape(NW * j_chunks, CH)

    n_out = -(-n // 128) * 128
    if n_out == n:
        n_out += 128
    n_acc = n_out

    zeros_big = jnp.zeros((n_acc, d), jnp.float32)
    n_rows_deg = -(-(n + 1) // 128)
    n_rows_deg += n_rows_deg % 2
    zerosf = jnp.zeros((n_rows_deg, 128), jnp.float32)

    r = 400
    assert n % r == 0
    grid = (n // r,)
    lw = [ln_w_0.reshape(1, d), ln_w_1.reshape(1, d), ln_w_2.reshape(1, d)]
    lb = [ln_b_0.reshape(1, d), ln_b_1.reshape(1, d), ln_b_2.reshape(1, d)]
    bias = [b_0.reshape(1, d), b_1.reshape(1, d), b_2.reshape(1, d)]
    Ws = [W_0, W_1, W_2]
    bp2 = bp.reshape(1, d)

    f32 = jnp.float32

    degp = _sc_deg(dst2d, zerosf, n_rows=n_rows_deg, j_chunks=j_chunks)
    deg_col = (degp[0] + degp[1]).reshape(n_rows_deg * 128, 1)[:n]

    zt0 = pl.pallas_call(
        _tc_a_body, grid=grid,
        in_specs=[_row_spec(r, d), _bc_spec(d), _bc_spec(d), _w_spec(d)],
        out_specs=_row_spec(r, d),
        out_shape=jax.ShapeDtypeStruct((n, d), f32),
    )(emb, lw[0], lb[0], Ws[0])

    ztp, dinv = pl.pallas_call(
        _tc_b_body, grid=grid,
        in_specs=[_row_spec(r, d), _row_spec(r, 1)],
        out_specs=[_row_spec(r, d), _row_spec(r, 1)],
        out_shape=[jax.ShapeDtypeStruct((n, d), f32),
                   jax.ShapeDtypeStruct((n, 1), f32)],
    )(zt0, deg_col)

    edge_k = _make_sc_edges(n_acc=n_acc, n_out=n_out, d=d, j_chunks=j_chunks,
                            active_core=1)
    h = emb
    for layer in range(3):
        aggp = edge_k(ztp, src2d, dst2d, zeros_big)
        if layer < 2:
            h, ztp = pl.pallas_call(
                _tc_c_body, grid=grid,
                in_specs=[_pair_spec(r, d), _row_spec(r, d), _row_spec(r, 1),
                          _bc_spec(d), _row_spec(r, d),
                          _bc_spec(d), _bc_spec(d), _w_spec(d)],
                out_specs=[_row_spec(r, d), _row_spec(r, d)],
                out_shape=[jax.ShapeDtypeStruct((n, d), f32),
                           jax.ShapeDtypeStruct((n, d), f32)],
            )(aggp, ztp, dinv, bias[layer], h,
              lw[layer + 1], lb[layer + 1], Ws[layer + 1])
        else:
            out = pl.pallas_call(
                _tc_d_body, grid=grid,
                in_specs=[_pair_spec(r, d), _row_spec(r, d), _row_spec(r, 1),
                          _bc_spec(d), _row_spec(r, d),
                          _w_spec(d), _bc_spec(d)],
                out_specs=_row_spec(r, d),
                out_shape=jax.ShapeDtypeStruct((n, d), f32),
            )(aggp, ztp, dinv, bias[layer], h, Wp, bp2)
    return out


# speedup vs baseline: 1.4502x; 1.4502x over previous
import dataclasses
import functools

import jax
import jax.numpy as jnp
from jax import lax
from jax.experimental import pallas as pl
from jax.experimental.pallas import tpu as pltpu
from jax.experimental.pallas import tpu_sc as plsc

NC = 2
NS = 16
NW = NC * NS
CH = 128

_SQRT2 = 1.4142135623730951


def _ln(x, w, b):
    mu = jnp.mean(x, axis=-1, keepdims=True)
    xc = x - mu
    var = jnp.mean(xc * xc, axis=-1, keepdims=True)
    return xc * lax.rsqrt(var + 1e-5) * w + b


def _mm_t(z, w_ref):
    return lax.dot_general(z, w_ref[...], (((1,), (1,)), ((), ())),
                           precision=lax.Precision.HIGHEST,
                           preferred_element_type=jnp.float32)


def _gelu_exact(x):
    return 0.5 * x * (1.0 + lax.erf(x / _SQRT2))


def _tc_a_body(h_ref, lw_ref, lb_ref, w_ref, zt_ref):
    z = _ln(h_ref[...], lw_ref[...], lb_ref[...])
    zt_ref[...] = _mm_t(z, w_ref)


def _tc_b_body(zt_ref, deg_ref, ztp_ref, dinv_ref):
    dv = lax.rsqrt(deg_ref[...] + 1.0)
    dinv_ref[...] = dv
    ztp_ref[...] = zt_ref[...] * dv


def _tc_c_body(aggp_ref, ztp_ref, dinv_ref, b_ref, h_ref,
               lw_ref, lb_ref, w_ref, hn_ref, ztpn_ref):
    dv = dinv_ref[...]
    s = (aggp_ref[0] + aggp_ref[1] + ztp_ref[...]) * dv + b_ref[...]
    hn = _gelu_exact(s) + h_ref[...]
    hn_ref[...] = hn
    z = _ln(hn, lw_ref[...], lb_ref[...])
    ztpn_ref[...] = _mm_t(z, w_ref) * dv


def _tc_d_body(aggp_ref, ztp_ref, dinv_ref, b_ref, h_ref,
               wp_ref, bp_ref, out_ref):
    dv = dinv_ref[...]
    s = (aggp_ref[0] + aggp_ref[1] + ztp_ref[...]) * dv + b_ref[...]
    hn = _gelu_exact(s) + h_ref[...]
    out_ref[...] = _mm_t(hn, wp_ref) + bp_ref[...]


def _row_spec(r, d):
    return pl.BlockSpec((r, d), lambda i: (i, 0))


def _bc_spec(d):
    return pl.BlockSpec((1, d), lambda i: (0, 0))


def _w_spec(d):
    return pl.BlockSpec((d, d), lambda i: (0, 0))


def _pair_spec(r, d):
    return pl.BlockSpec((2, r, d), lambda i: (0, i, 0))


def _sc_compiler_params(tc_tiling=True):
    cp = pltpu.CompilerParams()
    if "needs_layout_passes" in pltpu.CompilerParams.__dataclass_fields__:
        cp = dataclasses.replace(cp, needs_layout_passes=False)
    if not tc_tiling:
        cp = dataclasses.replace(cp, use_tc_tiling_on_sc=False)
    return cp


def _sc_deg(dst2d, zerosf, *, n_rows, j_chunks):
    assert n_rows <= 128
    mesh = plsc.VectorSubcoreMesh(core_axis_name="c", subcore_axis_name="s",
                                  num_cores=NC, num_subcores=NS)

    @functools.partial(
        pl.kernel, mesh=mesh, compiler_params=_sc_compiler_params(),
        out_type=jax.ShapeDtypeStruct((NC, n_rows, 128), jnp.float32),
        scratch_types=[
            pltpu.VMEM((j_chunks, CH), jnp.int32),
            pltpu.VMEM((n_rows, 128), jnp.float32),
            pltpu.VMEM((1, n_rows), jnp.int32),
            pltpu.VMEM_SHARED((n_rows, 128), jnp.float32),
        ],
    )
    def k(dst_hbm, zf_hbm, out_hbm, dst_v, acc2d, rowidx, accs):
        cid = lax.axis_index("c")
        sid = lax.axis_index("s")
        wid = sid * NC + cid
        pltpu.sync_copy(zf_hbm, acc2d)

        @pl.when(sid == 0)
        def _():
            pltpu.sync_copy(zf_hbm, accs)

        pltpu.sync_copy(dst_hbm.at[pl.ds(wid * j_chunks, j_chunks)], dst_v)

        @pl.loop(0, n_rows // 16)
        def _(c):
            rowidx[0, pl.ds(c * 16, 16)] = lax.iota(jnp.int32, 16) + c * 16

        ones = jnp.ones((16,), jnp.float32)

        @pl.loop(0, j_chunks)
        def _(j):
            @pl.loop(0, CH // 16)
            def _(c):
                idx = dst_v[j, pl.ds(c * 16, 16)]
                plsc.addupdate_scatter(
                    acc2d, [lax.shift_right_logical(idx, 7),
                            lax.bitwise_and(idx, 127)], ones)

        plsc.subcore_barrier()
        pltpu.sync_copy(acc2d, accs.at[rowidx.at[0]], add=True)
        plsc.subcore_barrier()
        half = (n_rows // 2 // 8) * 8

        @pl.when(sid == 0)
        def _():
            pltpu.sync_copy(accs.at[pl.ds(0, half)],
                            out_hbm.at[cid].at[pl.ds(0, half)])

        @pl.when(sid == 1)
        def _():
            pltpu.sync_copy(accs.at[pl.ds(half, n_rows - half)],
                            out_hbm.at[cid].at[pl.ds(half, n_rows - half)])

    return k(dst2d, zerosf)


def _make_sc_edges(*, n_acc, n_out, d, j_chunks, ce):
    rows_z = n_acc // NS
    rows_o = n_out // NS
    nh = 4
    jh = j_chunks // nh
    ng = d // 32
    mesh = plsc.VectorSubcoreMesh(core_axis_name="c", subcore_axis_name="s",
                                  num_cores=NC, num_subcores=NS)

    @functools.partial(
        pl.kernel, mesh=mesh,
        compiler_params=_sc_compiler_params(tc_tiling=False),
        out_type=jax.ShapeDtypeStruct((NC, n_out, d), jnp.float32),
        scratch_types=[
            pltpu.VMEM((jh, ce), jnp.int32),
            pltpu.VMEM((jh, ce), jnp.int32),
            pltpu.VMEM((ce, d // 2), jnp.int32),
            pltpu.VMEM((ce, d // 2), jnp.int32),
            pltpu.VMEM((ce, d), jnp.float32),
            pltpu.VMEM((ce, d), jnp.float32),
            pltpu.VMEM_SHARED((n_acc, d), jnp.float32),
            pltpu.SemaphoreType.DMA,
            pltpu.SemaphoreType.DMA,
            pltpu.SemaphoreType.DMA,
            pltpu.SemaphoreType.DMA,
        ],
    )
    def k(ztp_hbm, src_hbm, dst_hbm, zb_hbm, out_hbm,
          src_v, dst_v, gb0, gb1, cb0, cb1, acc, sg0, sg1, ss0, ss1):
        cid = lax.axis_index("c")
        sid = lax.axis_index("s")
        pltpu.sync_copy(zb_hbm.at[pl.ds(sid * rows_z, rows_z)],
                        acc.at[pl.ds(sid * rows_z, rows_z)])
        plsc.subcore_barrier()

        def convert(gb, cb):
            @pl.loop(0, ce)
            def _(r):
                for g in range(ng):
                    w = gb[r, pl.ds(g * 16, 16)]
                    lo = lax.shift_left(w, 16)
                    hi = lax.shift_left(lax.shift_right_logical(w, 16), 16)
                    cb[r, pl.ds(g * 32, 16)] = plsc.bitcast(lo, jnp.float32)
                    cb[r, pl.ds(g * 32 + 16, 16)] = plsc.bitcast(
                        hi, jnp.float32)

        for h in range(nh):
            base = (sid * NC + cid) * j_chunks + h * jh
            pltpu.sync_copy(src_hbm.at[pl.ds(base, jh)], src_v)
            pltpu.sync_copy(dst_hbm.at[pl.ds(base, jh)], dst_v)
            pltpu.async_copy(ztp_hbm.at[src_v.at[0]], gb0, sg0)
            pltpu.async_copy(ztp_hbm.at[src_v.at[1]], gb1, sg1)

            @pl.loop(0, jh // 2)
            def _(p):
                j0 = 2 * p
                j1 = 2 * p + 1
                for j, gb, cb, sg, ss in ((j0, gb0, cb0, sg0, ss0),
                                          (j1, gb1, cb1, sg1, ss1)):
                    pltpu.make_async_copy(ztp_hbm.at[src_v.at[j]],
                                          gb, sg).wait()

                    @pl.when(j >= 2)
                    def _():
                        pltpu.make_async_copy(
                            cb, acc.at[dst_v.at[j - 2]], ss).wait()

                    convert(gb, cb)
                    pltpu.async_copy(cb, acc.at[dst_v.at[j]], ss, add=True)

                    @pl.when(j + 2 < jh)
                    def _():
                        pltpu.async_copy(ztp_hbm.at[src_v.at[j + 2]], gb, sg)

            pltpu.make_async_copy(cb0, acc.at[dst_v.at[jh - 2]], ss0).wait()
            pltpu.make_async_copy(cb1, acc.at[dst_v.at[jh - 1]], ss1).wait()

        plsc.subcore_barrier()
        pltpu.sync_copy(acc.at[pl.ds(sid * rows_o, rows_o)],
                        out_hbm.at[cid].at[pl.ds(sid * rows_o, rows_o)])

    return k


def kernel(x, edge_index, emb,
           ln_w_0, ln_b_0, W_0, b_0,
           ln_w_1, ln_b_1, W_1, b_1,
           ln_w_2, ln_b_2, W_2, b_2,
           Wp, bp):
    n, d = emb.shape
    e = edge_index.shape[1]

    src = edge_index[0]
    dst = edge_index[1]
    j_chunks = -(-(-(-e // (NW * CH))) // 16) * 16
    e_pad = j_chunks * NW * CH
    pad = e_pad - e
    src_p, dst_p = src, dst
    if pad:
        src_p = jnp.concatenate([src, jnp.zeros((pad,), jnp.int32)])
        dst_p = jnp.concatenate([dst, jnp.full((pad,), n, jnp.int32)])
    dst2d = dst_p.reshape(NW * j_chunks, CH)

    ce = 64
    j_edges = -(-(-(-e // (NW * ce))) // 32) * 32
    e_pad_e = j_edges * NW * ce
    pad_e = e_pad_e - e
    src_pe, dst_pe = src, dst
    if pad_e:
        src_pe = jnp.concatenate([src, jnp.zeros((pad_e,), jnp.int32)])
        dst_pe = jnp.concatenate([dst, jnp.full((pad_e,), n, jnp.int32)])
    src2d_e = src_pe.reshape(NW * j_edges, ce)
    dst2d_e = dst_pe.reshape(NW * j_edges, ce)

    perm = []
    for g in range(d // 32):
        for i in range(16):
            perm.extend((g * 32 + i, g * 32 + 16 + i))
    perm = jnp.asarray(perm, jnp.int32)

    n_out = -(-n // 128) * 128
    if n_out == n:
        n_out += 128
    n_acc = n_out

    zeros_big = jnp.zeros((n_acc, d), jnp.float32)
    n_rows_deg = -(-(n + 1) // 128)
    n_rows_deg += n_rows_deg % 2
    zerosf = jnp.zeros((n_rows_deg, 128), jnp.float32)

    r = 400
    assert n % r == 0
    grid = (n // r,)
    lw = [ln_w_0.reshape(1, d), ln_w_1.reshape(1, d), ln_w_2.reshape(1, d)]
    lb = [ln_b_0.reshape(1, d), ln_b_1.reshape(1, d), ln_b_2.reshape(1, d)]
    bias = [b_0.reshape(1, d), b_1.reshape(1, d), b_2.reshape(1, d)]
    Ws = [W_0, W_1, W_2]
    bp2 = bp.reshape(1, d)

    f32 = jnp.float32

    degp = _sc_deg(dst2d, zerosf, n_rows=n_rows_deg, j_chunks=j_chunks)
    deg_col = (degp[0] + degp[1]).reshape(n_rows_deg * 128, 1)[:n]

    zt0 = pl.pallas_call(
        _tc_a_body, grid=grid,
        in_specs=[_row_spec(r, d), _bc_spec(d), _bc_spec(d), _w_spec(d)],
        out_specs=_row_spec(r, d),
        out_shape=jax.ShapeDtypeStruct((n, d), f32),
    )(emb, lw[0], lb[0], Ws[0])

    ztp, dinv = pl.pallas_call(
        _tc_b_body, grid=grid,
        in_specs=[_row_spec(r, d), _row_spec(r, 1)],
        out_specs=[_row_spec(r, d), _row_spec(r, 1)],
        out_shape=[jax.ShapeDtypeStruct((n, d), f32),
                   jax.ShapeDtypeStruct((n, 1), f32)],
    )(zt0, deg_col)

    edge_k = _make_sc_edges(n_acc=n_acc, n_out=n_out, d=d, j_chunks=j_edges,
                            ce=ce)
    h = emb
    for layer in range(3):
        ztp_b = ztp[:, perm].astype(jnp.bfloat16)
        ztp_b32 = lax.bitcast_convert_type(
            ztp_b.reshape(n, d // 2, 2), jnp.int32)
        aggp = edge_k(ztp_b32, src2d_e, dst2d_e, zeros_big)
        if layer < 2:
            h, ztp = pl.pallas_call(
                _tc_c_body, grid=grid,
                in_specs=[_pair_spec(r, d), _row_spec(r, d), _row_spec(r, 1),
                          _bc_spec(d), _row_spec(r, d),
                          _bc_spec(d), _bc_spec(d), _w_spec(d)],
                out_specs=[_row_spec(r, d), _row_spec(r, d)],
                out_shape=[jax.ShapeDtypeStruct((n, d), f32),
                           jax.ShapeDtypeStruct((n, d), f32)],
            )(aggp, ztp, dinv, bias[layer], h,
              lw[layer + 1], lb[layer + 1], Ws[layer + 1])
        else:
            out = pl.pallas_call(
                _tc_d_body, grid=grid,
                in_specs=[_pair_spec(r, d), _row_spec(r, d), _row_spec(r, 1),
                          _bc_spec(d), _row_spec(r, d),
                          _w_spec(d), _bc_spec(d)],
                out_specs=_row_spec(r, d),
                out_shape=jax.ShapeDtypeStruct((n, d), f32),
            )(aggp, ztp, dinv, bias[layer], h, Wp, bp2)
    return out
